# Initial kernel scaffold; baseline (speedup 1.0000x reference)
import jax
import jax.numpy as jnp
from jax.experimental import pallas as pl


def kernel(X, I):
    raise NotImplementedError("write your pallas kernel here")


# speedup vs baseline: 28.1320x; 28.1320x over previous
import functools

import jax
import jax.numpy as jnp
from jax import lax
from jax.experimental import pallas as pl
from jax.experimental.pallas import tpu as pltpu
from jax.experimental.pallas import tpu_sc as plsc

DIM_IN = 64
NUM_INDS = 16
N_BINS = 64
B = 8
N = 65536

NC = 2
NS = 16
NW = NC * NS

ROWS_TOTAL = B * N
ROWS_PER_W = ROWS_TOTAL // NW
CHUNK_ROWS = 2048
N_CHUNKS = ROWS_PER_W // CHUNK_ROWS
HIST_W = N_BINS + 1
UNROLL = 4


def _pos_body(x_ref, iw_ref, out_ref):
    s = lax.dot_general(x_ref[...], iw_ref[...],
                        (((1,), (1,)), ((), ())),
                        preferred_element_type=jnp.float32)
    v = jax.nn.sigmoid(s)
    out_ref[...] = jnp.clip(v * float(N_BINS) - 0.5, 0.0, float(N_BINS - 1))


def _compute_pos(Xf, Iw, blk_rows=4096):
    grid = (ROWS_TOTAL // blk_rows,)
    return pl.pallas_call(
        _pos_body,
        grid=grid,
        in_specs=[
            pl.BlockSpec((blk_rows, DIM_IN), lambda i: (i, 0)),
            pl.BlockSpec((NUM_INDS, DIM_IN), lambda i: (0, 0)),
        ],
        out_specs=pl.BlockSpec((blk_rows, NUM_INDS), lambda i: (i, 0)),
        out_shape=jax.ShapeDtypeStruct((ROWS_TOTAL, NUM_INDS), jnp.float32),
    )(Xf, Iw)


def _hist_body(pos_hbm, out_hbm, buf0, buf1, hist, sem0, sem1):
    wid = lax.axis_index("s") * NC + lax.axis_index("c")
    base = wid * (ROWS_PER_W * NUM_INDS)

    zeros16 = jnp.zeros((16,), jnp.float32)
    for i in range(HIST_W * NUM_INDS // 16):
        hist[pl.ds(i * 16, 16)] = zeros16
    lane_base = lax.iota(jnp.int32, 16) * HIST_W

    bufs = [buf0, buf1]
    sems = [sem0, sem1]
    cw = CHUNK_ROWS * NUM_INDS

    def _start(c):
        return pltpu.make_async_copy(
            pos_hbm.at[pl.ds(base + c * cw, cw)], bufs[c % 2], sems[c % 2]
        )

    _start(0).start()
    for c in range(N_CHUNKS):
        if c + 1 < N_CHUNKS:
            _start(c + 1).start()
        _start(c).wait()
        buf = bufs[c % 2]

        def body(it, carry):
            r0 = it * (UNROLL * NUM_INDS)
            for u in range(UNROLL):
                v = buf[pl.ds(r0 + u * NUM_INDS, 16)]
                i0 = v.astype(jnp.int32)
                frac = v - i0.astype(jnp.float32)
                idx0 = lane_base + i0
                plsc.addupdate_scatter(hist, [idx0], 1.0 - frac)
                plsc.addupdate_scatter(hist, [idx0 + 1], frac)
            return carry

        lax.fori_loop(0, CHUNK_ROWS // UNROLL, body, 0)

    pltpu.sync_copy(hist, out_hbm.at[pl.ds(wid * (NUM_INDS * HIST_W),
                                           NUM_INDS * HIST_W)])


_hist_call = functools.partial(
    pl.kernel,
    out_type=jax.ShapeDtypeStruct((NW * NUM_INDS * HIST_W,), jnp.float32),
    mesh=plsc.VectorSubcoreMesh(core_axis_name="c", subcore_axis_name="s"),
    scratch_types=[
        pltpu.VMEM((CHUNK_ROWS * NUM_INDS,), jnp.float32),
        pltpu.VMEM((CHUNK_ROWS * NUM_INDS,), jnp.float32),
        pltpu.VMEM((NUM_INDS * HIST_W,), jnp.float32),
        pltpu.SemaphoreType.DMA,
        pltpu.SemaphoreType.DMA,
    ],
    compiler_params=pltpu.CompilerParams(needs_layout_passes=False),
)(_hist_body)


def kernel(X, I):
    Xf = X.reshape(ROWS_TOTAL, DIM_IN)
    Iw = I[0]
    pos = _compute_pos(Xf, Iw)
    parts = _hist_call(pos.reshape(ROWS_TOTAL * NUM_INDS))
    parts = parts.reshape(B, NW // B, NUM_INDS, HIST_W)
    hist = parts.sum(axis=1)[:, :, :N_BINS] * (1.0 / N)
    return hist.reshape(B, NUM_INDS * N_BINS)


# speedup vs baseline: 38.2027x; 1.3580x over previous
import functools

import jax
import jax.numpy as jnp
from jax import lax
from jax.experimental import pallas as pl
from jax.experimental.pallas import tpu as pltpu
from jax.experimental.pallas import tpu_sc as plsc

DIM_IN = 64
NUM_INDS = 16
N_BINS = 64
B = 8
N = 65536

NC = 2
NS = 16
NW = NC * NS

ROWS_TOTAL = B * N
GROUP = 128 // NUM_INDS
G_ROWS = ROWS_TOTAL // GROUP
ROWS_PER_W = ROWS_TOTAL // NW
CHUNK_ROWS = 2048
N_CHUNKS = ROWS_PER_W // CHUNK_ROWS
HIST_W = N_BINS + 1
UNROLL = 8


def _pos_body(x_ref, w_ref, out_ref):
    s = lax.dot_general(x_ref[...], w_ref[...],
                        (((1,), (0,)), ((), ())),
                        preferred_element_type=jnp.float32)
    v = jax.nn.sigmoid(s)
    out_ref[...] = jnp.clip(v * float(N_BINS) - 0.5, 0.0, float(N_BINS - 1))


def _compute_pos(Xg, W, blk_rows=512):
    grid = (G_ROWS // blk_rows,)
    return pl.pallas_call(
        _pos_body,
        grid=grid,
        in_specs=[
            pl.BlockSpec((blk_rows, GROUP * DIM_IN), lambda i: (i, 0)),
            pl.BlockSpec((GROUP * DIM_IN, 128), lambda i: (0, 0)),
        ],
        out_specs=pl.BlockSpec((blk_rows, 128), lambda i: (i, 0)),
        out_shape=jax.ShapeDtypeStruct((G_ROWS, 128), jnp.float32),
    )(Xg, W)


def _hist_body(pos_hbm, out_hbm, buf0, buf1, hist, sem0, sem1):
    wid = lax.axis_index("s") * NC + lax.axis_index("c")
    base = wid * (ROWS_PER_W * NUM_INDS)

    zeros16 = jnp.zeros((16,), jnp.float32)
    for i in range(HIST_W * NUM_INDS // 16):
        hist[pl.ds(i * 16, 16)] = zeros16
    lane_base = lax.iota(jnp.int32, 16) * HIST_W

    bufs = [buf0, buf1]
    sems = [sem0, sem1]
    cw = CHUNK_ROWS * NUM_INDS

    def _copy(c):
        return pltpu.make_async_copy(
            pos_hbm.at[pl.ds(base + c * cw, cw)], bufs[c % 2], sems[c % 2]
        )

    _copy(0).start()
    for c in range(N_CHUNKS):
        if c + 1 < N_CHUNKS:
            _copy(c + 1).start()
        _copy(c).wait()
        buf = bufs[c % 2]

        @plsc.parallel_loop(0, CHUNK_ROWS, 1, unroll=UNROLL)
        def _row(r):
            v = buf[pl.ds(r * NUM_INDS, 16)]
            i0 = v.astype(jnp.int32)
            frac = v - i0.astype(jnp.float32)
            idx0 = lane_base + i0
            plsc.addupdate_scatter(hist, [idx0], 1.0 - frac)
            plsc.addupdate_scatter(hist, [idx0 + 1], frac)

    pltpu.sync_copy(hist, out_hbm.at[pl.ds(wid * (NUM_INDS * HIST_W),
                                           NUM_INDS * HIST_W)])


_hist_call = functools.partial(
    pl.kernel,
    out_type=jax.ShapeDtypeStruct((NW * NUM_INDS * HIST_W,), jnp.float32),
    mesh=plsc.VectorSubcoreMesh(core_axis_name="c", subcore_axis_name="s"),
    scratch_types=[
        pltpu.VMEM((CHUNK_ROWS * NUM_INDS,), jnp.float32),
        pltpu.VMEM((CHUNK_ROWS * NUM_INDS,), jnp.float32),
        pltpu.VMEM((NUM_INDS * HIST_W,), jnp.float32),
        pltpu.SemaphoreType.DMA,
        pltpu.SemaphoreType.DMA,
    ],
    compiler_params=pltpu.CompilerParams(needs_layout_passes=False),
)(_hist_body)


def kernel(X, I):
    Xg = X.reshape(G_ROWS, GROUP * DIM_IN)
    Iw = I[0]
    W = jnp.kron(jnp.eye(GROUP, dtype=jnp.float32), Iw.T)
    pos = _compute_pos(Xg, W)
    parts = _hist_call(pos.reshape(ROWS_TOTAL * NUM_INDS))
    parts = parts.reshape(B, NW // B, NUM_INDS, HIST_W)
    hist = parts.sum(axis=1)[:, :, :N_BINS] * (1.0 / N)
    return hist.reshape(B, NUM_INDS * N_BINS)


# speedup vs baseline: 89.7748x; 2.3500x over previous
import functools

import jax
import jax.numpy as jnp
from jax import lax
from jax.experimental import pallas as pl
from jax.experimental.pallas import tpu as pltpu
from jax.experimental.pallas import tpu_sc as plsc

DIM_IN = 64
NUM_INDS = 16
N_BINS = 64
B = 8
N = 65536

NC = 2
NS = 16
NW = NC * NS

N_ROWS = B * NUM_INDS
ROWS_PER_W = N_ROWS // NW
CHUNK_VALS = 32768
CHUNKS_PER_ROW = N // CHUNK_VALS
N_CHUNKS = ROWS_PER_W * CHUNKS_PER_ROW
HIST_W = 80
UNROLL = 8
NBLK = 4096


def _pos_body(iw_ref, x_ref, out_ref):
    s = lax.dot_general(iw_ref[...], x_ref[0],
                        (((1,), (0,)), ((), ())),
                        preferred_element_type=jnp.float32)
    v = jax.nn.sigmoid(s)
    out_ref[0] = jnp.clip(v * float(N_BINS) - 0.5, 0.0, float(N_BINS - 1))


def _compute_pos(Xt, Iw):
    grid = (B, N // NBLK)
    return pl.pallas_call(
        _pos_body,
        grid=grid,
        in_specs=[
            pl.BlockSpec((NUM_INDS, DIM_IN), lambda b, j: (0, 0)),
            pl.BlockSpec((1, DIM_IN, NBLK), lambda b, j: (b, 0, j)),
        ],
        out_specs=pl.BlockSpec((1, NUM_INDS, NBLK), lambda b, j: (b, 0, j)),
        out_shape=jax.ShapeDtypeStruct((B, NUM_INDS, N), jnp.float32),
    )(Iw, Xt)


def _hist_body(pos_hbm, out_hbm, buf0, buf1, hist, stage, sem0, sem1):
    wid = lax.axis_index("s") * NC + lax.axis_index("c")
    base = wid * (ROWS_PER_W * N)

    zeros16 = jnp.zeros((16,), jnp.float32)
    lane_base = lax.iota(jnp.int32, 16) * HIST_W

    bufs = [buf0, buf1]
    sems = [sem0, sem1]

    def _copy(c):
        return pltpu.make_async_copy(
            pos_hbm.at[pl.ds(base + c * CHUNK_VALS, CHUNK_VALS)],
            bufs[c % 2], sems[c % 2],
        )

    _copy(0).start()
    for c in range(N_CHUNKS):
        if c + 1 < N_CHUNKS:
            _copy(c + 1).start()
        if c % CHUNKS_PER_ROW == 0:
            for i in range(16 * HIST_W // 16):
                hist[pl.ds(i * 16, 16)] = zeros16
        _copy(c).wait()
        buf = bufs[c % 2]

        @plsc.parallel_loop(0, CHUNK_VALS // 16, 1, unroll=UNROLL)
        def _vec(r):
            v = buf[pl.ds(r * 16, 16)]
            i0 = v.astype(jnp.int32)
            frac = v - i0.astype(jnp.float32)
            idx0 = lane_base + i0
            plsc.addupdate_scatter(hist, [idx0], 1.0 - frac)
            plsc.addupdate_scatter(hist, [idx0 + 1], frac)

        if c % CHUNKS_PER_ROW == CHUNKS_PER_ROW - 1:
            row = c // CHUNKS_PER_ROW
            for g in range(HIST_W // 16):
                acc = hist[pl.ds(g * 16, 16)]
                for l in range(1, 16):
                    acc = acc + hist[pl.ds(l * HIST_W + g * 16, 16)]
                stage[pl.ds(row * HIST_W + g * 16, 16)] = acc

    pltpu.sync_copy(stage, out_hbm.at[pl.ds(wid * (ROWS_PER_W * HIST_W),
                                            ROWS_PER_W * HIST_W)])


_hist_call = functools.partial(
    pl.kernel,
    out_type=jax.ShapeDtypeStruct((N_ROWS * HIST_W,), jnp.float32),
    mesh=plsc.VectorSubcoreMesh(core_axis_name="c", subcore_axis_name="s"),
    scratch_types=[
        pltpu.VMEM((CHUNK_VALS,), jnp.float32),
        pltpu.VMEM((CHUNK_VALS,), jnp.float32),
        pltpu.VMEM((16 * HIST_W,), jnp.float32),
        pltpu.VMEM((ROWS_PER_W * HIST_W,), jnp.float32),
        pltpu.SemaphoreType.DMA,
        pltpu.SemaphoreType.DMA,
    ],
    compiler_params=pltpu.CompilerParams(needs_layout_passes=False),
)(_hist_body)


def kernel(X, I):
    Xt = X.transpose(0, 2, 1)
    Iw = I[0]
    pos = _compute_pos(Xt, Iw)
    hist = _hist_call(pos.reshape(B * NUM_INDS * N))
    hist = hist.reshape(B, NUM_INDS, HIST_W)[:, :, :N_BINS] * (1.0 / N)
    return hist.reshape(B, NUM_INDS * N_BINS)
